# Initial kernel scaffold; baseline (speedup 1.0000x reference)
import jax
import jax.numpy as jnp
from jax.experimental import pallas as pl


def kernel(scores, k):
    raise NotImplementedError("write your pallas kernel here")


# speedup vs baseline: 19.4834x; 19.4834x over previous
import jax
import jax.numpy as jnp
from jax import lax
from jax.experimental import pallas as pl
from jax.experimental.pallas import tpu as pltpu
from jax.experimental.pallas import tpu_sc as plsc

NC, NS, L = 2, 16, 16
NW = NC * NS
ROWS, COLS = 128, 32768
RPW = ROWS // NW
K = 8
GROUP = 16
NGRP = COLS // (GROUP * L)
CAP = 2048
NEG = -100000.0
IMAX = 2**31 - 1


def _body(scores_hbm, kofs_hbm, masked_hbm, vals_hbm, idx_hbm,
          row_v, cvals_v, cidx_v, gmax_v, kofs_v, stg_f, stg_i):
    wid = lax.axis_index("s") * NC + lax.axis_index("c")
    pltpu.sync_copy(kofs_hbm, kofs_v)
    kofs = jnp.max(kofs_v[...])
    lanes = lax.broadcasted_iota(jnp.int32, (L,), 0)
    ninf = jnp.float32(-jnp.inf)
    ninf_v = jnp.full((L,), ninf, jnp.float32)
    neg_v = jnp.full((L,), NEG, jnp.float32)

    def do_row(r, _):
        row = wid * RPW + r
        pltpu.sync_copy(scores_hbm.at[row], row_v)

        def grp1(g, lm):
            gm = ninf_v
            for j in range(GROUP):
                gm = jnp.maximum(gm, row_v[pl.ds((g * GROUP + j) * L, L)])
            gmax_v[pl.ds(g * L, L)] = gm
            return jnp.maximum(lm, gm)
        lm = lax.fori_loop(0, NGRP, grp1, ninf_v)

        t = ninf
        for _i in range(K):
            t = jnp.max(lm)
            lm = jnp.where(lm == t, ninf_v, lm)

        def clr(j, _c):
            cvals_v[pl.ds(j * L, L)] = ninf_v
            return 0
        lax.fori_loop(0, (CAP + L) // L, clr, 0)

        def grp2(g, off):
            gsm = jnp.max(gmax_v[pl.ds(g * L, L)])

            def collect(off):
                for j in range(GROUP):
                    base = (g * GROUP + j) * L
                    v = row_v[pl.ds(base, L)]
                    m = v >= t
                    cnt = jnp.sum(m.astype(jnp.int32))
                    o = jnp.minimum(off, CAP)
                    plsc.store_compressed(cvals_v.at[pl.ds(o, L)], v, mask=m)
                    plsc.store_compressed(cidx_v.at[pl.ds(o, L)],
                                          lanes + base, mask=m)
                    off = jnp.minimum(off + cnt, CAP)
                return off
            return lax.cond(gsm >= t, collect, lambda o: o, off)
        used = lax.fori_loop(0, NGRP, grp2, jnp.int32(0))
        nv = (used + L - 1) // L

        def round_fn(i, carry):
            tv, ti, _v8 = carry

            def amax(jv, m):
                return jnp.maximum(m, cvals_v[pl.ds(jv * L, L)])
            mx = jnp.max(lax.fori_loop(0, nv, amax, ninf_v))

            def amin(jv, mi):
                cv = cvals_v[pl.ds(jv * L, L)]
                ci = cidx_v[pl.ds(jv * L, L)]
                return jnp.minimum(mi, jnp.where(cv == mx, ci, IMAX))
            mix = jnp.min(lax.fori_loop(0, nv, amin,
                                        jnp.full((L,), IMAX, jnp.int32)))

            def rem(jv, _c):
                cv = cvals_v[pl.ds(jv * L, L)]
                ci = cidx_v[pl.ds(jv * L, L)]
                cvals_v[pl.ds(jv * L, L)] = jnp.where(ci == mix, ninf_v, cv)
                return 0
            lax.fori_loop(0, nv, rem, 0)
            tv = jnp.where(lanes == i, mx, tv)
            ti = jnp.where(lanes == i, mix, ti)
            return tv, ti, mx
        tv, ti, v8 = lax.fori_loop(
            0, K, round_fn,
            (ninf_v, jnp.zeros((L,), jnp.int32), ninf))

        stg_f[...] = tv + kofs
        stg_i[...] = ti
        pltpu.sync_copy(stg_f.at[pl.ds(0, K)], vals_hbm.at[pl.ds(row * K, K)])
        pltpu.sync_copy(stg_i.at[pl.ds(0, K)], idx_hbm.at[pl.ds(row * K, K)])

        def grp3(g, _c):
            gsm = jnp.max(gmax_v[pl.ds(g * L, L)])

            def fill(_o):
                for j in range(GROUP):
                    row_v[pl.ds((g * GROUP + j) * L, L)] = neg_v
                return 0

            def maskg(_o):
                for j in range(GROUP):
                    base = (g * GROUP + j) * L
                    v = row_v[pl.ds(base, L)]
                    row_v[pl.ds(base, L)] = jnp.where(v >= v8, v + kofs, neg_v)
                return 0
            lax.cond(gsm >= v8, maskg, fill, 0)
            return 0
        lax.fori_loop(0, NGRP, grp3, 0)
        pltpu.sync_copy(row_v, masked_hbm.at[row])
        return 0

    lax.fori_loop(0, RPW, do_row, 0)


def kernel(scores, k):
    kofs = jnp.full((L,), 1.0, jnp.float32) * (
        jnp.asarray(k, jnp.int32) - K).astype(jnp.float32)
    mesh = plsc.VectorSubcoreMesh(core_axis_name="c", subcore_axis_name="s",
                                  num_cores=NC, num_subcores=NS)
    f = pl.kernel(
        _body,
        out_type=[
            jax.ShapeDtypeStruct((ROWS, COLS), jnp.float32),
            jax.ShapeDtypeStruct((ROWS * K,), jnp.float32),
            jax.ShapeDtypeStruct((ROWS * K,), jnp.int32),
        ],
        mesh=mesh,
        compiler_params=pltpu.CompilerParams(needs_layout_passes=False),
        scratch_types=[
            pltpu.VMEM((COLS,), jnp.float32),
            pltpu.VMEM((CAP + L,), jnp.float32),
            pltpu.VMEM((CAP + L,), jnp.int32),
            pltpu.VMEM((NGRP * L,), jnp.float32),
            pltpu.VMEM((L,), jnp.float32),
            pltpu.VMEM((L,), jnp.float32),
            pltpu.VMEM((L,), jnp.int32),
        ],
    )
    masked, vals, idx = f(scores, kofs)
    return masked, vals.reshape(ROWS, K), idx.reshape(ROWS, K)


# speedup vs baseline: 21.5648x; 1.1068x over previous
import jax
import jax.numpy as jnp
from jax import lax
from jax.experimental import pallas as pl
from jax.experimental.pallas import tpu as pltpu
from jax.experimental.pallas import tpu_sc as plsc

NC, NS, L = 2, 16, 16
NW = NC * NS
ROWS, COLS = 128, 32768
RPW = ROWS // NW
K = 8
GROUP = 16
NGRP = COLS // (GROUP * L)
CAP = 2048
NEG = -100000.0
IMAX = 2**31 - 1
NBUF = 3


def _body(scores_hbm, kofs_hbm, masked_hbm, vals_hbm, idx_hbm,
          row0_v, row1_v, row2_v, cvals_v, cidx_v, gmax_v, kofs_v,
          pack_f, pack_i,
          sin0, sin1, sin2, sout0, sout1, sout2):
    rowbufs = [row0_v, row1_v, row2_v]
    sin = [sin0, sin1, sin2]
    sout = [sout0, sout1, sout2]
    wid = lax.axis_index("s") * NC + lax.axis_index("c")
    pltpu.sync_copy(kofs_hbm, kofs_v)
    kofs = jnp.max(kofs_v[...])
    lanes = lax.broadcasted_iota(jnp.int32, (L,), 0)
    ninf = jnp.float32(-jnp.inf)
    ninf_v = jnp.full((L,), ninf, jnp.float32)
    neg_v = jnp.full((L,), NEG, jnp.float32)
    row_base = wid * RPW

    in_h = [None] * RPW
    out_h = [None] * RPW
    out_waited = set()
    in_h[0] = pltpu.async_copy(scores_hbm.at[row_base], rowbufs[0], sin[0])

    tvpack = ninf_v
    tipack = jnp.zeros((L,), jnp.int32)

    for r in range(RPW):
        buf = rowbufs[r % NBUF]
        row = row_base + r
        in_h[r].wait()
        if r + 1 < RPW:
            if r + 1 >= NBUF:
                out_h[r + 1 - NBUF].wait()
                out_waited.add(r + 1 - NBUF)
            in_h[r + 1] = pltpu.async_copy(
                scores_hbm.at[row + 1], rowbufs[(r + 1) % NBUF],
                sin[(r + 1) % NBUF])

        def grp1(g, lm, buf=buf):
            acc = [ninf_v, ninf_v, ninf_v, ninf_v]
            for j in range(GROUP):
                acc[j % 4] = jnp.maximum(
                    acc[j % 4], buf[pl.ds((g * GROUP + j) * L, L)])
            gm = jnp.maximum(jnp.maximum(acc[0], acc[1]),
                             jnp.maximum(acc[2], acc[3]))
            gmax_v[pl.ds(g * L, L)] = gm
            return jnp.maximum(lm, gm)
        lm = lax.fori_loop(0, NGRP, grp1, ninf_v)

        t = ninf
        for _i in range(K):
            t = jnp.max(lm)
            lm = jnp.where(lm == t, ninf_v, lm)

        def grp2(g, off, buf=buf, t=t):
            gsm = jnp.max(gmax_v[pl.ds(g * L, L)])

            def collect(off):
                for j in range(GROUP):
                    base = (g * GROUP + j) * L
                    v = buf[pl.ds(base, L)]
                    m = v >= t
                    cnt = jnp.sum(m.astype(jnp.int32))
                    o = jnp.minimum(off, CAP)
                    plsc.store_compressed(cvals_v.at[pl.ds(o, L)], v, mask=m)
                    plsc.store_compressed(cidx_v.at[pl.ds(o, L)],
                                          lanes + base, mask=m)
                    off = jnp.minimum(off + cnt, CAP)
                return off
            return lax.cond(gsm >= t, collect, lambda o: o, off)
        used = lax.fori_loop(0, NGRP, grp2, jnp.int32(0))
        nv = (used + L - 1) // L
        cvals_v[pl.ds(used, L)] = ninf_v

        lane_base = (r % 2) * K

        def round_fn(i, carry, nv=nv, lane_base=lane_base):
            tv, ti, _v8 = carry

            def amax(jv, m):
                return jnp.maximum(m, cvals_v[pl.ds(jv * L, L)])
            mx = jnp.max(lax.fori_loop(0, nv, amax, ninf_v))

            def amin(jv, mi):
                cv = cvals_v[pl.ds(jv * L, L)]
                ci = cidx_v[pl.ds(jv * L, L)]
                return jnp.minimum(mi, jnp.where(cv == mx, ci, IMAX))
            mix = jnp.min(lax.fori_loop(0, nv, amin,
                                        jnp.full((L,), IMAX, jnp.int32)))

            def rem(jv, _c):
                cv = cvals_v[pl.ds(jv * L, L)]
                ci = cidx_v[pl.ds(jv * L, L)]
                cvals_v[pl.ds(jv * L, L)] = jnp.where(ci == mix, ninf_v, cv)
                return 0
            lax.fori_loop(0, nv, rem, 0)
            tv = jnp.where(lanes == lane_base + i, mx, tv)
            ti = jnp.where(lanes == lane_base + i, mix, ti)
            return tv, ti, mx
        tvpack, tipack, v8 = lax.fori_loop(
            0, K, round_fn, (tvpack, tipack, ninf))
        if r % 2 == 1:
            pack_f[pl.ds((r // 2) * L, L)] = tvpack + kofs
            pack_i[pl.ds((r // 2) * L, L)] = tipack
            tvpack = ninf_v
            tipack = jnp.zeros((L,), jnp.int32)

        def grp3(g, _c, buf=buf, v8=v8):
            gsm = jnp.max(gmax_v[pl.ds(g * L, L)])

            def fill(_o):
                for j in range(GROUP):
                    buf[pl.ds((g * GROUP + j) * L, L)] = neg_v
                return 0

            def maskg(_o):
                for j in range(GROUP):
                    base = (g * GROUP + j) * L
                    v = buf[pl.ds(base, L)]
                    buf[pl.ds(base, L)] = jnp.where(v >= v8, v + kofs, neg_v)
                return 0
            lax.cond(gsm >= v8, maskg, fill, 0)
            return 0
        lax.fori_loop(0, NGRP, grp3, 0)

        out_h[r] = pltpu.async_copy(buf, masked_hbm.at[row], sout[r % NBUF])

    for r in range(RPW):
        if r not in out_waited:
            out_h[r].wait()

    pltpu.sync_copy(pack_f, vals_hbm.at[pl.ds(wid * RPW * K, RPW * K)])
    pltpu.sync_copy(pack_i, idx_hbm.at[pl.ds(wid * RPW * K, RPW * K)])


def kernel(scores, k):
    kofs = jnp.full((L,), 1.0, jnp.float32) * (
        jnp.asarray(k, jnp.int32) - K).astype(jnp.float32)
    mesh = plsc.VectorSubcoreMesh(core_axis_name="c", subcore_axis_name="s",
                                  num_cores=NC, num_subcores=NS)
    f = pl.kernel(
        _body,
        out_type=[
            jax.ShapeDtypeStruct((ROWS, COLS), jnp.float32),
            jax.ShapeDtypeStruct((ROWS * K,), jnp.float32),
            jax.ShapeDtypeStruct((ROWS * K,), jnp.int32),
        ],
        mesh=mesh,
        compiler_params=pltpu.CompilerParams(needs_layout_passes=False),
        scratch_types=[
            pltpu.VMEM((COLS,), jnp.float32),
            pltpu.VMEM((COLS,), jnp.float32),
            pltpu.VMEM((COLS,), jnp.float32),
            pltpu.VMEM((CAP + L,), jnp.float32),
            pltpu.VMEM((CAP + L,), jnp.int32),
            pltpu.VMEM((NGRP * L,), jnp.float32),
            pltpu.VMEM((L,), jnp.float32),
            pltpu.VMEM((RPW * K,), jnp.float32),
            pltpu.VMEM((RPW * K,), jnp.int32),
            pltpu.SemaphoreType.DMA,
            pltpu.SemaphoreType.DMA,
            pltpu.SemaphoreType.DMA,
            pltpu.SemaphoreType.DMA,
            pltpu.SemaphoreType.DMA,
            pltpu.SemaphoreType.DMA,
        ],
    )
    masked, vals, idx = f(scores, kofs)
    return masked, vals.reshape(ROWS, K), idx.reshape(ROWS, K)
